# Initial kernel scaffold; baseline (speedup 1.0000x reference)
import jax
import jax.numpy as jnp
from jax.experimental import pallas as pl


def kernel(node_features, edge_index, edge_features, node_proj_w, node_proj_b, edge_nn_w, edge_nn_b, conv_bias_0, conv_bias_1):
    raise NotImplementedError("write your pallas kernel here")


# speedup vs baseline: 3.6781x; 3.6781x over previous
import functools

import jax
import jax.numpy as jnp
import numpy as np
from jax import lax
from jax.experimental import pallas as pl
from jax.experimental.pallas import tpu as pltpu
from jax.experimental.pallas import tpu_sc as plsc

N = 10000
E = 320000
IN_FEATS = 128
EDGE_FEATS = 4
HIDDEN = 16
HH = HIDDEN * HIDDEN

NC = 2
NS = 16
NW = NC * NS
GW = 128
WPW = 80
E_PAD = NW * WPW * GW
N_SP = 10016
RPT = N_SP // NS
CH = 8

@functools.cache
def _mesh():
    return plsc.VectorSubcoreMesh(
        core_axis_name="c", subcore_axis_name="s", num_cores=NC, num_subcores=NS
    )


def _sc_gather(table, idx2d):
    @functools.partial(
        pl.kernel,
        out_type=jax.ShapeDtypeStruct((E_PAD, HIDDEN), jnp.float32),
        mesh=_mesh(),
        compiler_params=pltpu.CompilerParams(use_tc_tiling_on_sc=False),
    )
    def kern(tab_hbm, idx_hbm, out_hbm):
        def body(i_vmem, o_vmem):
            pltpu.sync_copy(tab_hbm.at[i_vmem.at[0]], o_vmem)

        pltpu.emit_pipeline(
            body,
            grid=(E_PAD // GW,),
            in_specs=[pl.BlockSpec((1, GW), lambda i: (0, i))],
            out_specs=[pl.BlockSpec((GW, HIDDEN), lambda i: (i, 0))],
            core_axis_name=("c", "s"),
            dimension_semantics=(pltpu.PARALLEL,),
        )(idx_hbm, out_hbm)

    return kern(table, idx2d)


@functools.cache
def _make_sc_scatter(with_deg):
    n_out = 2 if with_deg else 1
    out_type = tuple(
        jax.ShapeDtypeStruct((NC * N_SP, HIDDEN), jnp.float32) for _ in range(n_out)
    )
    if n_out == 1:
        out_type = out_type[0]

    scratch_types = [
        pltpu.VMEM((WPW, GW), jnp.int32),
        pltpu.VMEM((CH * GW, HIDDEN), jnp.float32),
        pltpu.VMEM((GW, HIDDEN), jnp.float32),
        pltpu.VMEM_SHARED((N_SP, HIDDEN), jnp.float32),
        pltpu.VMEM_SHARED((N_SP, HIDDEN), jnp.float32),
    ]

    @functools.partial(
        pl.kernel, out_type=out_type, mesh=_mesh(), scratch_types=scratch_types,
        compiler_params=pltpu.CompilerParams(use_tc_tiling_on_sc=False),
    )
    def kern(msg_hbm, dst_hbm, aux_hbm, *refs):
        if with_deg:
            agg_out, deg_out, idx_v, msg_v, ones_v, agg_sp, deg_sp = refs
        else:
            agg_out, idx_v, msg_v, ones_v, agg_sp, deg_sp = refs
        c = lax.axis_index("c")
        s = lax.axis_index("s")
        wid = c * NS + s

        pltpu.sync_copy(aux_hbm.at[pl.ds(0, RPT)], agg_sp.at[pl.ds(s * RPT, RPT)])
        if with_deg:
            pltpu.sync_copy(aux_hbm.at[pl.ds(0, RPT)], deg_sp.at[pl.ds(s * RPT, RPT)])
            pltpu.sync_copy(aux_hbm.at[pl.ds(RPT, GW)], ones_v)
        pltpu.sync_copy(dst_hbm.at[pl.ds(wid * WPW, WPW)], idx_v)
        plsc.subcore_barrier()

        @pl.loop(0, WPW // CH)
        def _(t):
            row0 = (wid * WPW + t * CH) * GW
            pltpu.sync_copy(msg_hbm.at[pl.ds(row0, CH * GW)], msg_v)
            for j in range(CH):
                pltpu.sync_copy(
                    msg_v.at[pl.ds(j * GW, GW)],
                    agg_sp.at[idx_v.at[t * CH + j]],
                    add=True,
                )
                if with_deg:
                    pltpu.sync_copy(
                        ones_v, deg_sp.at[idx_v.at[t * CH + j]], add=True
                    )

        plsc.subcore_barrier()
        pltpu.sync_copy(
            agg_sp.at[pl.ds(s * RPT, RPT)],
            agg_out.at[pl.ds(c * N_SP + s * RPT, RPT)],
        )
        if with_deg:
            pltpu.sync_copy(
                deg_sp.at[pl.ds(s * RPT, RPT)],
                deg_out.at[pl.ds(c * N_SP + s * RPT, RPT)],
            )

    return kern


def _tc_proj(x, w, b):
    def body(x_ref, w_ref, b_ref, o_ref):
        o_ref[...] = (
            jnp.dot(x_ref[...], w_ref[...], preferred_element_type=jnp.float32)
            + b_ref[...]
        )

    return pl.pallas_call(
        body, out_shape=jax.ShapeDtypeStruct((N, HIDDEN), jnp.float32)
    )(x, w, b.reshape(1, HIDDEN))


_BE = 4096


def _tc_msg(ef, hs, wnn, bnn, S, T):
    def body(ef_ref, hs_ref, w_ref, b_ref, s_ref, t_ref, o_ref):
        ew = jnp.maximum(
            jnp.dot(ef_ref[...], w_ref[...], preferred_element_type=jnp.float32)
            + b_ref[...],
            0.0,
        )
        rep = jnp.dot(hs_ref[...], s_ref[...], preferred_element_type=jnp.float32)
        o_ref[...] = jnp.dot(rep * ew, t_ref[...], preferred_element_type=jnp.float32)

    return pl.pallas_call(
        body,
        grid=(E_PAD // _BE,),
        in_specs=[
            pl.BlockSpec((_BE, EDGE_FEATS), lambda i: (i, 0)),
            pl.BlockSpec((_BE, HIDDEN), lambda i: (i, 0)),
            pl.BlockSpec((EDGE_FEATS, HH), lambda i: (0, 0)),
            pl.BlockSpec((1, HH), lambda i: (0, 0)),
            pl.BlockSpec((HIDDEN, HH), lambda i: (0, 0)),
            pl.BlockSpec((HH, HIDDEN), lambda i: (0, 0)),
        ],
        out_specs=pl.BlockSpec((_BE, HIDDEN), lambda i: (i, 0)),
        out_shape=jax.ShapeDtypeStruct((E_PAD, HIDDEN), jnp.float32),
    )(ef, hs, wnn, bnn, S, T)


def _tc_norm1(a0, a1, d0, d1, b):
    def body(a0_ref, a1_ref, d0_ref, d1_ref, b_ref, h_ref, inv_ref):
        inv = 1.0 / jnp.maximum(d0_ref[...] + d1_ref[...], 1.0)
        inv_ref[...] = inv
        h_ref[...] = jnp.maximum((a0_ref[...] + a1_ref[...]) * inv + b_ref[...], 0.0)

    return pl.pallas_call(
        body,
        out_shape=(
            jax.ShapeDtypeStruct((N, HIDDEN), jnp.float32),
            jax.ShapeDtypeStruct((N, HIDDEN), jnp.float32),
        ),
    )(a0, a1, d0, d1, b.reshape(1, HIDDEN))


def _tc_norm2(a0, a1, inv, b):
    def body(a0_ref, a1_ref, inv_ref, b_ref, h_ref):
        h_ref[...] = jnp.maximum(
            (a0_ref[...] + a1_ref[...]) * inv_ref[...] + b_ref[...], 0.0
        )

    return pl.pallas_call(
        body, out_shape=jax.ShapeDtypeStruct((N, HIDDEN), jnp.float32)
    )(a0, a1, inv, b.reshape(1, HIDDEN))


_S_NP = np.kron(np.eye(HIDDEN, dtype=np.float32), np.ones((1, HIDDEN), np.float32))
_T_NP = np.tile(np.eye(HIDDEN, dtype=np.float32), (HIDDEN, 1))


def kernel(node_features, edge_index, edge_features, node_proj_w, node_proj_b,
           edge_nn_w, edge_nn_b, conv_bias_0, conv_bias_1):
    pad = E_PAD - E
    src = jnp.concatenate([edge_index[0], jnp.zeros((pad,), jnp.int32)])
    src = src.reshape(1, E_PAD)
    dst = jnp.concatenate([edge_index[1], jnp.full((pad,), N, jnp.int32)])
    dst = dst.reshape(NW * WPW, GW)
    ef = jnp.concatenate(
        [edge_features, jnp.zeros((pad, EDGE_FEATS), jnp.float32)], axis=0
    )
    aux = jnp.concatenate(
        [jnp.zeros((RPT, HIDDEN), jnp.float32), jnp.ones((GW, HIDDEN), jnp.float32)]
    )
    S = jnp.asarray(_S_NP)
    T = jnp.asarray(_T_NP)
    bnn = edge_nn_b.reshape(1, HH)

    h = _tc_proj(node_features, node_proj_w, node_proj_b)

    hs = _sc_gather(h, src)
    msg = _tc_msg(ef, hs, edge_nn_w, bnn, S, T)
    agg_p, deg_p = _make_sc_scatter(True)(msg, dst, aux)
    h, invdeg = _tc_norm1(
        agg_p[:N], agg_p[N_SP : N_SP + N], deg_p[:N], deg_p[N_SP : N_SP + N],
        conv_bias_0,
    )

    hs = _sc_gather(h, src)
    msg = _tc_msg(ef, hs, edge_nn_w, bnn, S, T)
    agg_p = _make_sc_scatter(False)(msg, dst, aux)
    h = _tc_norm2(agg_p[:N], agg_p[N_SP : N_SP + N], invdeg, conv_bias_1)
    return h


# speedup vs baseline: 3.9344x; 1.0697x over previous
import functools

import jax
import jax.numpy as jnp
import numpy as np
from jax import lax
from jax.experimental import pallas as pl
from jax.experimental.pallas import tpu as pltpu
from jax.experimental.pallas import tpu_sc as plsc

N = 10000
E = 320000
IN_FEATS = 128
EDGE_FEATS = 4
HIDDEN = 16
HH = HIDDEN * HIDDEN

NC = 2
NS = 16
NW = NC * NS
GW = 128
WPW = 80
E_PAD = NW * WPW * GW
N_SP = 10016
RPT = N_SP // NS
CH = 8

@functools.cache
def _mesh():
    return plsc.VectorSubcoreMesh(
        core_axis_name="c", subcore_axis_name="s", num_cores=NC, num_subcores=NS
    )


def _sc_gather(table, idx2d):
    @functools.partial(
        pl.kernel,
        out_type=jax.ShapeDtypeStruct((E_PAD, HIDDEN), jnp.float32),
        mesh=_mesh(),
        scratch_types=[
            pltpu.VMEM((WPW, GW), jnp.int32),
            pltpu.VMEM((CH * GW, HIDDEN), jnp.float32),
            pltpu.VMEM((CH * GW, HIDDEN), jnp.float32),
            pltpu.SemaphoreType.DMA,
            pltpu.SemaphoreType.DMA,
        ],
        compiler_params=pltpu.CompilerParams(use_tc_tiling_on_sc=False),
    )
    def kern(tab_hbm, idx_hbm, out_hbm, idx_v, gbuf0, gbuf1, g_sem, w_sem):
        gbufs = (gbuf0, gbuf1)
        c = lax.axis_index("c")
        s = lax.axis_index("s")
        wid = c * NS + s
        pltpu.sync_copy(idx_hbm.at[pl.ds(wid * WPW, WPW)], idx_v)
        n_chunks = WPW // CH
        wbs = [None] * n_chunks
        for t in range(n_chunks):
            b = t % 2
            if t >= 2:
                wbs[t - 2].wait()
            gs = [
                pltpu.async_copy(
                    tab_hbm.at[idx_v.at[t * CH + j]],
                    gbufs[b].at[pl.ds(j * GW, GW)],
                    g_sem,
                )
                for j in range(CH)
            ]
            for g in gs:
                g.wait()
            wbs[t] = pltpu.async_copy(
                gbufs[b],
                out_hbm.at[pl.ds((wid * WPW + t * CH) * GW, CH * GW)],
                w_sem,
            )
        wbs[n_chunks - 2].wait()
        wbs[n_chunks - 1].wait()

    return kern(table, idx2d)


@functools.cache
def _make_sc_scatter(with_deg):
    n_out = 2 if with_deg else 1
    out_type = tuple(
        jax.ShapeDtypeStruct((NC * N_SP, HIDDEN), jnp.float32) for _ in range(n_out)
    )
    if n_out == 1:
        out_type = out_type[0]

    scratch_types = [
        pltpu.VMEM((WPW, GW), jnp.int32),
        pltpu.VMEM((CH * GW, HIDDEN), jnp.float32),
        pltpu.VMEM((CH * GW, HIDDEN), jnp.float32),
        pltpu.VMEM((GW, HIDDEN), jnp.float32),
        pltpu.VMEM_SHARED((N_SP, HIDDEN), jnp.float32),
        pltpu.VMEM_SHARED((N_SP, HIDDEN), jnp.float32),
        pltpu.SemaphoreType.DMA,
    ]

    @functools.partial(
        pl.kernel, out_type=out_type, mesh=_mesh(), scratch_types=scratch_types,
        compiler_params=pltpu.CompilerParams(use_tc_tiling_on_sc=False),
    )
    def kern(msg_hbm, dst_hbm, aux_hbm, *refs):
        if with_deg:
            agg_out, deg_out, idx_v, mv0, mv1, ones_v, agg_sp, deg_sp, m_sem = refs
        else:
            agg_out, idx_v, mv0, mv1, ones_v, agg_sp, deg_sp, m_sem = refs
        msg_bufs = (mv0, mv1)
        c = lax.axis_index("c")
        s = lax.axis_index("s")
        wid = c * NS + s

        pltpu.sync_copy(aux_hbm.at[pl.ds(0, RPT)], agg_sp.at[pl.ds(s * RPT, RPT)])
        if with_deg:
            pltpu.sync_copy(aux_hbm.at[pl.ds(0, RPT)], deg_sp.at[pl.ds(s * RPT, RPT)])
            pltpu.sync_copy(aux_hbm.at[pl.ds(RPT, GW)], ones_v)
        pltpu.sync_copy(dst_hbm.at[pl.ds(wid * WPW, WPW)], idx_v)
        plsc.subcore_barrier()

        n_chunks = WPW // CH

        def fetch(t, b):
            return pltpu.async_copy(
                msg_hbm.at[pl.ds((wid * WPW + t * CH) * GW, CH * GW)],
                msg_bufs[b],
                m_sem,
            )

        ld = fetch(0, 0)
        for t in range(n_chunks):
            b = t % 2
            ld.wait()
            if t + 1 < n_chunks:
                ld = fetch(t + 1, 1 - b)
            for j in range(CH):
                pltpu.sync_copy(
                    msg_bufs[b].at[pl.ds(j * GW, GW)],
                    agg_sp.at[idx_v.at[t * CH + j]],
                    add=True,
                )
                if with_deg:
                    pltpu.sync_copy(
                        ones_v, deg_sp.at[idx_v.at[t * CH + j]], add=True
                    )

        plsc.subcore_barrier()
        pltpu.sync_copy(
            agg_sp.at[pl.ds(s * RPT, RPT)],
            agg_out.at[pl.ds(c * N_SP + s * RPT, RPT)],
        )
        if with_deg:
            pltpu.sync_copy(
                deg_sp.at[pl.ds(s * RPT, RPT)],
                deg_out.at[pl.ds(c * N_SP + s * RPT, RPT)],
            )

    return kern


def _tc_proj(x, w, b):
    def body(x_ref, w_ref, b_ref, o_ref):
        o_ref[...] = (
            jnp.dot(x_ref[...], w_ref[...], preferred_element_type=jnp.float32)
            + b_ref[...]
        )

    return pl.pallas_call(
        body, out_shape=jax.ShapeDtypeStruct((N, HIDDEN), jnp.float32)
    )(x, w, b.reshape(1, HIDDEN))


_BE = 2560
_N_REAL_BLK = E // _BE


def _tc_msg(ef, hs, wnn, bnn, S, T):
    def body(ef_ref, hs_ref, w_ref, b_ref, s_ref, t_ref, o_ref):
        ew = jnp.maximum(
            jnp.dot(ef_ref[...], w_ref[...], preferred_element_type=jnp.float32)
            + b_ref[...],
            0.0,
        )
        rep = jnp.dot(hs_ref[...], s_ref[...], preferred_element_type=jnp.float32)
        o_ref[...] = jnp.dot(rep * ew, t_ref[...], preferred_element_type=jnp.float32)

    return pl.pallas_call(
        body,
        grid=(E_PAD // _BE,),
        in_specs=[
            pl.BlockSpec(
                (_BE, EDGE_FEATS),
                lambda i: (jnp.minimum(i, _N_REAL_BLK - 1), 0),
            ),
            pl.BlockSpec((_BE, HIDDEN), lambda i: (i, 0)),
            pl.BlockSpec((EDGE_FEATS, HH), lambda i: (0, 0)),
            pl.BlockSpec((1, HH), lambda i: (0, 0)),
            pl.BlockSpec((HIDDEN, HH), lambda i: (0, 0)),
            pl.BlockSpec((HH, HIDDEN), lambda i: (0, 0)),
        ],
        out_specs=pl.BlockSpec((_BE, HIDDEN), lambda i: (i, 0)),
        out_shape=jax.ShapeDtypeStruct((E_PAD, HIDDEN), jnp.float32),
    )(ef, hs, wnn, bnn, S, T)


def _tc_norm1(a0, a1, d0, d1, b):
    def body(a0_ref, a1_ref, d0_ref, d1_ref, b_ref, h_ref, inv_ref):
        inv = 1.0 / jnp.maximum(d0_ref[...] + d1_ref[...], 1.0)
        inv_ref[...] = inv
        h_ref[...] = jnp.maximum((a0_ref[...] + a1_ref[...]) * inv + b_ref[...], 0.0)

    return pl.pallas_call(
        body,
        out_shape=(
            jax.ShapeDtypeStruct((N, HIDDEN), jnp.float32),
            jax.ShapeDtypeStruct((N, HIDDEN), jnp.float32),
        ),
    )(a0, a1, d0, d1, b.reshape(1, HIDDEN))


def _tc_norm2(a0, a1, inv, b):
    def body(a0_ref, a1_ref, inv_ref, b_ref, h_ref):
        h_ref[...] = jnp.maximum(
            (a0_ref[...] + a1_ref[...]) * inv_ref[...] + b_ref[...], 0.0
        )

    return pl.pallas_call(
        body, out_shape=jax.ShapeDtypeStruct((N, HIDDEN), jnp.float32)
    )(a0, a1, inv, b.reshape(1, HIDDEN))


_S_NP = np.kron(np.eye(HIDDEN, dtype=np.float32), np.ones((1, HIDDEN), np.float32))
_T_NP = np.tile(np.eye(HIDDEN, dtype=np.float32), (HIDDEN, 1))


def kernel(node_features, edge_index, edge_features, node_proj_w, node_proj_b,
           edge_nn_w, edge_nn_b, conv_bias_0, conv_bias_1):
    pad = E_PAD - E
    src = jnp.concatenate([edge_index[0], jnp.zeros((pad,), jnp.int32)])
    src = src.reshape(NW * WPW, GW)
    dst = jnp.concatenate([edge_index[1], jnp.full((pad,), N, jnp.int32)])
    dst = dst.reshape(NW * WPW, GW)
    ef = edge_features
    aux = jnp.concatenate(
        [jnp.zeros((RPT, HIDDEN), jnp.float32), jnp.ones((GW, HIDDEN), jnp.float32)]
    )
    S = jnp.asarray(_S_NP)
    T = jnp.asarray(_T_NP)
    bnn = edge_nn_b.reshape(1, HH)

    h = _tc_proj(node_features, node_proj_w, node_proj_b)

    hs = _sc_gather(h, src)
    msg = _tc_msg(ef, hs, edge_nn_w, bnn, S, T)
    agg_p, deg_p = _make_sc_scatter(True)(msg, dst, aux)
    h, invdeg = _tc_norm1(
        agg_p[:N], agg_p[N_SP : N_SP + N], deg_p[:N], deg_p[N_SP : N_SP + N],
        conv_bias_0,
    )

    hs = _sc_gather(h, src)
    msg = _tc_msg(ef, hs, edge_nn_w, bnn, S, T)
    agg_p = _make_sc_scatter(False)(msg, dst, aux)
    h = _tc_norm2(agg_p[:N], agg_p[N_SP : N_SP + N], invdeg, conv_bias_1)
    return h


# speedup vs baseline: 5.6070x; 1.4251x over previous
import functools

import jax
import jax.numpy as jnp
import numpy as np
from jax import lax
from jax.experimental import pallas as pl
from jax.experimental.pallas import tpu as pltpu
from jax.experimental.pallas import tpu_sc as plsc

N = 10000
E = 320000
IN_FEATS = 128
EDGE_FEATS = 4
HIDDEN = 16
HH = HIDDEN * HIDDEN

NC = 2
NS = 16
NW = NC * NS
GW = 128
WPW = 80
E_PAD = NW * WPW * GW
N_SP = 10016
RPT = N_SP // NS
CH = 8

@functools.cache
def _mesh():
    return plsc.VectorSubcoreMesh(
        core_axis_name="c", subcore_axis_name="s", num_cores=NC, num_subcores=NS
    )


def _sc_gather(table, idx2d):
    @functools.partial(
        pl.kernel,
        out_type=jax.ShapeDtypeStruct((E_PAD, HIDDEN), jnp.float32),
        mesh=_mesh(),
        scratch_types=[
            pltpu.VMEM((WPW, GW), jnp.int32),
            pltpu.VMEM((CH * GW, HIDDEN), jnp.float32),
            pltpu.VMEM((CH * GW, HIDDEN), jnp.float32),
            pltpu.SemaphoreType.DMA,
            pltpu.SemaphoreType.DMA,
        ],
        compiler_params=pltpu.CompilerParams(use_tc_tiling_on_sc=False),
    )
    def kern(tab_hbm, idx_hbm, out_hbm, idx_v, gbuf0, gbuf1, g_sem, w_sem):
        gbufs = (gbuf0, gbuf1)
        c = lax.axis_index("c")
        s = lax.axis_index("s")
        wid = c * NS + s
        pltpu.sync_copy(idx_hbm.at[pl.ds(wid * WPW, WPW)], idx_v)
        n_chunks = WPW // CH
        wbs = [None] * n_chunks
        for t in range(n_chunks):
            b = t % 2
            if t >= 2:
                wbs[t - 2].wait()
            gs = [
                pltpu.async_copy(
                    tab_hbm.at[idx_v.at[t * CH + j]],
                    gbufs[b].at[pl.ds(j * GW, GW)],
                    g_sem,
                )
                for j in range(CH)
            ]
            for g in gs:
                g.wait()
            wbs[t] = pltpu.async_copy(
                gbufs[b],
                out_hbm.at[pl.ds((wid * WPW + t * CH) * GW, CH * GW)],
                w_sem,
            )
        wbs[n_chunks - 2].wait()
        wbs[n_chunks - 1].wait()

    return kern(table, idx2d)


@functools.cache
def _make_sc_scatter(with_deg):
    n_out = 2 if with_deg else 1
    out_type = tuple(
        jax.ShapeDtypeStruct((NC * N_SP, HIDDEN), jnp.float32) for _ in range(n_out)
    )
    if n_out == 1:
        out_type = out_type[0]

    scratch_types = [
        pltpu.VMEM((WPW, GW), jnp.int32),
        pltpu.VMEM((CH * GW, HIDDEN), jnp.float32),
        pltpu.VMEM((CH * GW, HIDDEN), jnp.float32),
        pltpu.VMEM((GW, HIDDEN), jnp.float32),
        pltpu.VMEM_SHARED((N_SP, HIDDEN), jnp.float32),
        pltpu.VMEM_SHARED((N_SP, HIDDEN), jnp.float32),
        pltpu.SemaphoreType.DMA,
    ]

    @functools.partial(
        pl.kernel, out_type=out_type, mesh=_mesh(), scratch_types=scratch_types,
        compiler_params=pltpu.CompilerParams(use_tc_tiling_on_sc=False),
    )
    def kern(msg_hbm, dst_hbm, aux_hbm, *refs):
        if with_deg:
            agg_out, deg_out, idx_v, mv0, mv1, ones_v, agg_sp, deg_sp, m_sem = refs
        else:
            agg_out, idx_v, mv0, mv1, ones_v, agg_sp, deg_sp, m_sem = refs
        msg_bufs = (mv0, mv1)
        c = lax.axis_index("c")
        s = lax.axis_index("s")
        wid = c * NS + s

        pltpu.sync_copy(aux_hbm.at[pl.ds(0, RPT)], agg_sp.at[pl.ds(s * RPT, RPT)])
        if with_deg:
            pltpu.sync_copy(aux_hbm.at[pl.ds(0, RPT)], deg_sp.at[pl.ds(s * RPT, RPT)])
            pltpu.sync_copy(aux_hbm.at[pl.ds(RPT, GW)], ones_v)
        pltpu.sync_copy(dst_hbm.at[pl.ds(wid * WPW, WPW)], idx_v)
        plsc.subcore_barrier()

        n_chunks = WPW // CH

        def fetch(t, b):
            return pltpu.async_copy(
                msg_hbm.at[pl.ds((wid * WPW + t * CH) * GW, CH * GW)],
                msg_bufs[b],
                m_sem,
            )

        ld = fetch(0, 0)
        for t in range(n_chunks):
            b = t % 2
            ld.wait()
            if t + 1 < n_chunks:
                ld = fetch(t + 1, 1 - b)
            for j in range(CH):
                pltpu.sync_copy(
                    msg_bufs[b].at[pl.ds(j * GW, GW)],
                    agg_sp.at[idx_v.at[t * CH + j]],
                    add=True,
                )
                if with_deg:
                    pltpu.sync_copy(
                        ones_v, deg_sp.at[idx_v.at[t * CH + j]], add=True
                    )

        plsc.subcore_barrier()
        pltpu.sync_copy(
            agg_sp.at[pl.ds(s * RPT, RPT)],
            agg_out.at[pl.ds(c * N_SP + s * RPT, RPT)],
        )
        if with_deg:
            pltpu.sync_copy(
                deg_sp.at[pl.ds(s * RPT, RPT)],
                deg_out.at[pl.ds(c * N_SP + s * RPT, RPT)],
            )

    return kern


def _tc_proj(x, w, b):
    def body(x_ref, w_ref, b_ref, o_ref):
        o_ref[...] = (
            jnp.dot(x_ref[...], w_ref[...], preferred_element_type=jnp.float32)
            + b_ref[...]
        )

    return pl.pallas_call(
        body, out_shape=jax.ShapeDtypeStruct((N, HIDDEN), jnp.float32)
    )(x, w, b.reshape(1, HIDDEN))


_BE = 2560
_N_REAL_BLK = E // _BE


_PK = 8
_BR = _BE // _PK
_PW = _PK * HH


def _tc_msg(ef8, hs_p, w8, b8, S8, T8):
    def body(ef_ref, hs_ref, w_ref, b_ref, s_ref, t_ref, o_ref):
        ew = jnp.maximum(
            jnp.dot(ef_ref[...], w_ref[...], preferred_element_type=jnp.float32)
            + b_ref[...],
            0.0,
        )
        rep = jnp.dot(
            hs_ref[...].astype(jnp.bfloat16), s_ref[...],
            preferred_element_type=jnp.float32,
        )
        prod = (rep * ew).astype(jnp.bfloat16)
        o_ref[...] = jnp.dot(prod, t_ref[...], preferred_element_type=jnp.float32)

    return pl.pallas_call(
        body,
        grid=(E_PAD // _BE,),
        in_specs=[
            pl.BlockSpec(
                (_BR, _PK * EDGE_FEATS),
                lambda i: (jnp.minimum(i, _N_REAL_BLK - 1), 0),
            ),
            pl.BlockSpec((_BR, 128), lambda i: (i, 0)),
            pl.BlockSpec((_PK * EDGE_FEATS, _PW), lambda i: (0, 0)),
            pl.BlockSpec((1, _PW), lambda i: (0, 0)),
            pl.BlockSpec((128, _PW), lambda i: (0, 0)),
            pl.BlockSpec((_PW, 128), lambda i: (0, 0)),
        ],
        out_specs=pl.BlockSpec((_BR, 128), lambda i: (i, 0)),
        out_shape=jax.ShapeDtypeStruct((E_PAD * HIDDEN // 128, 128), jnp.float32),
    )(ef8, hs_p, w8, b8, S8, T8)


def _tc_norm1(a0, a1, d0, d1, b):
    def body(a0_ref, a1_ref, d0_ref, d1_ref, b_ref, h_ref, inv_ref):
        inv = 1.0 / jnp.maximum(d0_ref[...] + d1_ref[...], 1.0)
        inv_ref[...] = inv
        h_ref[...] = jnp.maximum((a0_ref[...] + a1_ref[...]) * inv + b_ref[...], 0.0)

    return pl.pallas_call(
        body,
        out_shape=(
            jax.ShapeDtypeStruct((N, HIDDEN), jnp.float32),
            jax.ShapeDtypeStruct((N, HIDDEN), jnp.float32),
        ),
    )(a0, a1, d0, d1, b.reshape(1, HIDDEN))


def _tc_norm2(a0, a1, inv, b):
    def body(a0_ref, a1_ref, inv_ref, b_ref, h_ref):
        h_ref[...] = jnp.maximum(
            (a0_ref[...] + a1_ref[...]) * inv_ref[...] + b_ref[...], 0.0
        )

    return pl.pallas_call(
        body, out_shape=jax.ShapeDtypeStruct((N, HIDDEN), jnp.float32)
    )(a0, a1, inv, b.reshape(1, HIDDEN))


_S_NP = np.kron(np.eye(HIDDEN, dtype=np.float32), np.ones((1, HIDDEN), np.float32))
_T_NP = np.tile(np.eye(HIDDEN, dtype=np.float32), (HIDDEN, 1))
_S8_NP = np.kron(np.eye(_PK, dtype=np.float32), _S_NP)
_T8_NP = np.kron(np.eye(_PK, dtype=np.float32), _T_NP)


def kernel(node_features, edge_index, edge_features, node_proj_w, node_proj_b,
           edge_nn_w, edge_nn_b, conv_bias_0, conv_bias_1):
    pad = E_PAD - E
    src = jnp.concatenate([edge_index[0], jnp.zeros((pad,), jnp.int32)])
    src = src.reshape(NW * WPW, GW)
    dst = jnp.concatenate([edge_index[1], jnp.full((pad,), N, jnp.int32)])
    dst = dst.reshape(NW * WPW, GW)
    ef8 = edge_features.reshape(E // _PK, _PK * EDGE_FEATS)
    aux = jnp.concatenate(
        [jnp.zeros((RPT, HIDDEN), jnp.float32), jnp.ones((GW, HIDDEN), jnp.float32)]
    )
    S8 = jnp.asarray(_S8_NP, dtype=jnp.bfloat16)
    T8 = jnp.asarray(_T8_NP, dtype=jnp.bfloat16)
    w8 = jnp.kron(jnp.eye(_PK, dtype=jnp.float32), edge_nn_w)
    b8 = jnp.tile(edge_nn_b, _PK).reshape(1, _PW)

    h = _tc_proj(node_features, node_proj_w, node_proj_b)

    def edge_stage(h_tab):
        hs_p = _sc_gather(h_tab, src).reshape(E_PAD * HIDDEN // 128, 128)
        msg_p = _tc_msg(ef8, hs_p, w8, b8, S8, T8)
        return msg_p.reshape(E_PAD, HIDDEN)

    msg = edge_stage(h)
    agg_p, deg_p = _make_sc_scatter(True)(msg, dst, aux)
    h, invdeg = _tc_norm1(
        agg_p[:N], agg_p[N_SP : N_SP + N], deg_p[:N], deg_p[N_SP : N_SP + N],
        conv_bias_0,
    )

    msg = edge_stage(h)
    agg_p = _make_sc_scatter(False)(msg, dst, aux)
    h = _tc_norm2(agg_p[:N], agg_p[N_SP : N_SP + N], invdeg, conv_bias_1)
    return h


# speedup vs baseline: 6.2473x; 1.1142x over previous
import functools

import jax
import jax.numpy as jnp
import numpy as np
from jax import lax
from jax.experimental import pallas as pl
from jax.experimental.pallas import tpu as pltpu
from jax.experimental.pallas import tpu_sc as plsc

N = 10000
E = 320000
IN_FEATS = 128
EDGE_FEATS = 4
HIDDEN = 16
HH = HIDDEN * HIDDEN

NC = 2
NS = 16
NW = NC * NS
GW = 128
WPW = 80
E_PAD = NW * WPW * GW
N_SP = 10016
RPT = N_SP // NS
CH = 8

@functools.cache
def _mesh():
    return plsc.VectorSubcoreMesh(
        core_axis_name="c", subcore_axis_name="s", num_cores=NC, num_subcores=NS
    )


def _sc_gather(table, idx2d):
    @functools.partial(
        pl.kernel,
        out_type=jax.ShapeDtypeStruct((E_PAD, HIDDEN), jnp.float32),
        mesh=_mesh(),
        scratch_types=[
            pltpu.VMEM((WPW, GW), jnp.int32),
            pltpu.VMEM((CH * GW, HIDDEN), jnp.float32),
            pltpu.VMEM((CH * GW, HIDDEN), jnp.float32),
            pltpu.SemaphoreType.DMA,
            pltpu.SemaphoreType.DMA,
        ],
        compiler_params=pltpu.CompilerParams(use_tc_tiling_on_sc=False),
    )
    def kern(tab_hbm, idx_hbm, out_hbm, idx_v, gbuf0, gbuf1, g_sem, w_sem):
        gbufs = (gbuf0, gbuf1)
        c = lax.axis_index("c")
        s = lax.axis_index("s")
        wid = c * NS + s
        pltpu.sync_copy(idx_hbm.at[pl.ds(wid * WPW, WPW)], idx_v)
        n_chunks = WPW // CH
        wbs = [None] * n_chunks
        for t in range(n_chunks):
            b = t % 2
            if t >= 2:
                wbs[t - 2].wait()
            gs = [
                pltpu.async_copy(
                    tab_hbm.at[idx_v.at[t * CH + j]],
                    gbufs[b].at[pl.ds(j * GW, GW)],
                    g_sem,
                )
                for j in range(CH)
            ]
            for g in gs:
                g.wait()
            wbs[t] = pltpu.async_copy(
                gbufs[b],
                out_hbm.at[pl.ds((wid * WPW + t * CH) * GW, CH * GW)],
                w_sem,
            )
        wbs[n_chunks - 2].wait()
        wbs[n_chunks - 1].wait()

    return kern(table, idx2d)


@functools.cache
def _make_sc_scatter(with_deg):
    n_out = 2 if with_deg else 1
    out_type = tuple(
        jax.ShapeDtypeStruct((NC * N_SP, HIDDEN), jnp.float32) for _ in range(n_out)
    )
    if n_out == 1:
        out_type = out_type[0]

    scratch_types = [
        pltpu.VMEM((WPW, GW), jnp.int32),
        pltpu.VMEM((CH * GW, HIDDEN), jnp.float32),
        pltpu.VMEM((CH * GW, HIDDEN), jnp.float32),
        pltpu.VMEM((GW, HIDDEN), jnp.float32),
        pltpu.VMEM_SHARED((N_SP, HIDDEN), jnp.float32),
        pltpu.VMEM_SHARED((N_SP, HIDDEN), jnp.float32),
        pltpu.SemaphoreType.DMA,
    ]

    @functools.partial(
        pl.kernel, out_type=out_type, mesh=_mesh(), scratch_types=scratch_types,
        compiler_params=pltpu.CompilerParams(use_tc_tiling_on_sc=False),
    )
    def kern(msg_hbm, dst_hbm, aux_hbm, *refs):
        if with_deg:
            agg_out, deg_out, idx_v, mv0, mv1, ones_v, agg_sp, deg_sp, m_sem = refs
        else:
            agg_out, idx_v, mv0, mv1, ones_v, agg_sp, deg_sp, m_sem = refs
        msg_bufs = (mv0, mv1)
        c = lax.axis_index("c")
        s = lax.axis_index("s")
        wid = c * NS + s

        pltpu.sync_copy(aux_hbm.at[pl.ds(0, RPT)], agg_sp.at[pl.ds(s * RPT, RPT)])
        if with_deg:
            pltpu.sync_copy(aux_hbm.at[pl.ds(0, RPT)], deg_sp.at[pl.ds(s * RPT, RPT)])
            pltpu.sync_copy(aux_hbm.at[pl.ds(RPT, GW)], ones_v)
        pltpu.sync_copy(dst_hbm.at[pl.ds(wid * WPW, WPW)], idx_v)
        plsc.subcore_barrier()

        n_chunks = WPW // CH

        def fetch(t, b):
            return pltpu.async_copy(
                msg_hbm.at[pl.ds((wid * WPW + t * CH) * GW, CH * GW)],
                msg_bufs[b],
                m_sem,
            )

        ld = fetch(0, 0)
        for t in range(n_chunks):
            b = t % 2
            ld.wait()
            if t + 1 < n_chunks:
                ld = fetch(t + 1, 1 - b)
            for j in range(CH):
                pltpu.sync_copy(
                    msg_bufs[b].at[pl.ds(j * GW, GW)],
                    agg_sp.at[idx_v.at[t * CH + j]],
                    add=True,
                )
                if with_deg:
                    pltpu.sync_copy(
                        ones_v, deg_sp.at[idx_v.at[t * CH + j]], add=True
                    )

        plsc.subcore_barrier()
        pltpu.sync_copy(
            agg_sp.at[pl.ds(s * RPT, RPT)],
            agg_out.at[pl.ds(c * N_SP + s * RPT, RPT)],
        )
        if with_deg:
            pltpu.sync_copy(
                deg_sp.at[pl.ds(s * RPT, RPT)],
                deg_out.at[pl.ds(c * N_SP + s * RPT, RPT)],
            )

    return kern


def _tc_proj(x, w, b):
    def body(x_ref, w_ref, b_ref, o_ref):
        o_ref[...] = (
            jnp.dot(x_ref[...], w_ref[...], preferred_element_type=jnp.float32)
            + b_ref[...]
        )

    return pl.pallas_call(
        body, out_shape=jax.ShapeDtypeStruct((N, HIDDEN), jnp.float32)
    )(x, w, b.reshape(1, HIDDEN))


_BE = 2560
_N_REAL_BLK = E // _BE


_PK = 8
_BR = _BE // _PK
_PW = _PK * HH


def _tc_msg(ef8, hs_p, w8, b8, S8, T8):
    def body(ef_ref, hs_ref, w_ref, b_ref, s_ref, t_ref, o_ref):
        ew_u = jnp.maximum(
            jnp.dot(ef_ref[...], w_ref[...], preferred_element_type=jnp.float32)
            + b_ref[...],
            0.0,
        )
        ew = ew_u.reshape(_BR, _PW)
        rep = jnp.dot(
            hs_ref[...].astype(jnp.bfloat16), s_ref[...],
            preferred_element_type=jnp.float32,
        )
        prod = (rep * ew).astype(jnp.bfloat16)
        o_ref[...] = jnp.dot(prod, t_ref[...], preferred_element_type=jnp.float32)

    return pl.pallas_call(
        body,
        grid=(E_PAD // _BE,),
        in_specs=[
            pl.BlockSpec(
                (_BE, EDGE_FEATS),
                lambda i: (jnp.minimum(i, _N_REAL_BLK - 1), 0),
            ),
            pl.BlockSpec((_BR, 128), lambda i: (i, 0)),
            pl.BlockSpec((EDGE_FEATS, HH), lambda i: (0, 0)),
            pl.BlockSpec((1, HH), lambda i: (0, 0)),
            pl.BlockSpec((128, _PW), lambda i: (0, 0)),
            pl.BlockSpec((_PW, 128), lambda i: (0, 0)),
        ],
        out_specs=pl.BlockSpec((_BR, 128), lambda i: (i, 0)),
        out_shape=jax.ShapeDtypeStruct((E_PAD * HIDDEN // 128, 128), jnp.float32),
    )(ef8, hs_p, w8, b8, S8, T8)


def _tc_norm1(a0, a1, d0, d1, b):
    def body(a0_ref, a1_ref, d0_ref, d1_ref, b_ref, h_ref, inv_ref):
        inv = 1.0 / jnp.maximum(d0_ref[...] + d1_ref[...], 1.0)
        inv_ref[...] = inv
        h_ref[...] = jnp.maximum((a0_ref[...] + a1_ref[...]) * inv + b_ref[...], 0.0)

    return pl.pallas_call(
        body,
        out_shape=(
            jax.ShapeDtypeStruct((N, HIDDEN), jnp.float32),
            jax.ShapeDtypeStruct((N, HIDDEN), jnp.float32),
        ),
    )(a0, a1, d0, d1, b.reshape(1, HIDDEN))


def _tc_norm2(a0, a1, inv, b):
    def body(a0_ref, a1_ref, inv_ref, b_ref, h_ref):
        h_ref[...] = jnp.maximum(
            (a0_ref[...] + a1_ref[...]) * inv_ref[...] + b_ref[...], 0.0
        )

    return pl.pallas_call(
        body, out_shape=jax.ShapeDtypeStruct((N, HIDDEN), jnp.float32)
    )(a0, a1, inv, b.reshape(1, HIDDEN))


_S_NP = np.kron(np.eye(HIDDEN, dtype=np.float32), np.ones((1, HIDDEN), np.float32))
_T_NP = np.tile(np.eye(HIDDEN, dtype=np.float32), (HIDDEN, 1))
_S8_NP = np.kron(np.eye(_PK, dtype=np.float32), _S_NP)
_T8_NP = np.kron(np.eye(_PK, dtype=np.float32), _T_NP)


def kernel(node_features, edge_index, edge_features, node_proj_w, node_proj_b,
           edge_nn_w, edge_nn_b, conv_bias_0, conv_bias_1):
    pad = E_PAD - E
    src = jnp.concatenate([edge_index[0], jnp.zeros((pad,), jnp.int32)])
    src = src.reshape(NW * WPW, GW)
    dst = jnp.concatenate([edge_index[1], jnp.full((pad,), N, jnp.int32)])
    dst = dst.reshape(NW * WPW, GW)
    aux = jnp.concatenate(
        [jnp.zeros((RPT, HIDDEN), jnp.float32), jnp.ones((GW, HIDDEN), jnp.float32)]
    )
    S8 = jnp.asarray(_S8_NP, dtype=jnp.bfloat16)
    T8 = jnp.asarray(_T8_NP, dtype=jnp.bfloat16)
    bnn = edge_nn_b.reshape(1, HH)

    h = _tc_proj(node_features, node_proj_w, node_proj_b)

    def edge_stage(h_tab):
        hs_p = _sc_gather(h_tab, src).reshape(E_PAD * HIDDEN // 128, 128)
        msg_p = _tc_msg(edge_features, hs_p, edge_nn_w, bnn, S8, T8)
        return msg_p.reshape(E_PAD, HIDDEN)

    msg = edge_stage(h)
    agg_p, deg_p = _make_sc_scatter(True)(msg, dst, aux)
    h, invdeg = _tc_norm1(
        agg_p[:N], agg_p[N_SP : N_SP + N], deg_p[:N], deg_p[N_SP : N_SP + N],
        conv_bias_0,
    )

    msg = edge_stage(h)
    agg_p = _make_sc_scatter(False)(msg, dst, aux)
    h = _tc_norm2(agg_p[:N], agg_p[N_SP : N_SP + N], invdeg, conv_bias_1)
    return h


# speedup vs baseline: 6.2942x; 1.0075x over previous
import functools

import jax
import jax.numpy as jnp
import numpy as np
from jax import lax
from jax.experimental import pallas as pl
from jax.experimental.pallas import tpu as pltpu
from jax.experimental.pallas import tpu_sc as plsc

N = 10000
E = 320000
IN_FEATS = 128
EDGE_FEATS = 4
HIDDEN = 16
HH = HIDDEN * HIDDEN

NC = 2
NS = 16
NW = NC * NS
GW = 128
WPW = 80
E_PAD = NW * WPW * GW
N_SP = 10016
RPT = N_SP // NS
CH = 8

@functools.cache
def _mesh():
    return plsc.VectorSubcoreMesh(
        core_axis_name="c", subcore_axis_name="s", num_cores=NC, num_subcores=NS
    )


def _sc_gather(table, idx2d):
    @functools.partial(
        pl.kernel,
        out_type=jax.ShapeDtypeStruct((E_PAD, HIDDEN), jnp.float32),
        mesh=_mesh(),
        scratch_types=[
            pltpu.VMEM((WPW, GW), jnp.int32),
            pltpu.VMEM((CH * GW, HIDDEN), jnp.float32),
            pltpu.VMEM((CH * GW, HIDDEN), jnp.float32),
            pltpu.SemaphoreType.DMA,
            pltpu.SemaphoreType.DMA,
        ],
        compiler_params=pltpu.CompilerParams(use_tc_tiling_on_sc=False),
    )
    def kern(tab_hbm, idx_hbm, out_hbm, idx_v, gbuf0, gbuf1, g_sem, w_sem):
        gbufs = (gbuf0, gbuf1)
        c = lax.axis_index("c")
        s = lax.axis_index("s")
        wid = c * NS + s
        pltpu.sync_copy(idx_hbm.at[pl.ds(wid * WPW, WPW)], idx_v)
        n_chunks = WPW // CH
        wbs = [None] * n_chunks
        for t in range(n_chunks):
            b = t % 2
            if t >= 2:
                wbs[t - 2].wait()
            gs = [
                pltpu.async_copy(
                    tab_hbm.at[idx_v.at[t * CH + j]],
                    gbufs[b].at[pl.ds(j * GW, GW)],
                    g_sem,
                )
                for j in range(CH)
            ]
            for g in gs:
                g.wait()
            wbs[t] = pltpu.async_copy(
                gbufs[b],
                out_hbm.at[pl.ds((wid * WPW + t * CH) * GW, CH * GW)],
                w_sem,
            )
        wbs[n_chunks - 2].wait()
        wbs[n_chunks - 1].wait()

    return kern(table, idx2d)


@functools.cache
def _make_sc_scatter(with_deg):
    n_out = 2 if with_deg else 1
    out_type = tuple(
        jax.ShapeDtypeStruct((NC * N_SP, HIDDEN), jnp.float32) for _ in range(n_out)
    )
    if n_out == 1:
        out_type = out_type[0]

    scratch_types = [
        pltpu.VMEM((WPW, GW), jnp.int32),
        pltpu.VMEM((CH * GW, HIDDEN), jnp.float32),
        pltpu.VMEM((CH * GW, HIDDEN), jnp.float32),
        pltpu.VMEM((GW, HIDDEN), jnp.float32),
        pltpu.VMEM_SHARED((N_SP, HIDDEN), jnp.float32),
        pltpu.VMEM_SHARED((N_SP, HIDDEN), jnp.float32),
        pltpu.SemaphoreType.DMA,
    ]

    @functools.partial(
        pl.kernel, out_type=out_type, mesh=_mesh(), scratch_types=scratch_types,
        compiler_params=pltpu.CompilerParams(use_tc_tiling_on_sc=False),
    )
    def kern(msg_hbm, dst_hbm, aux_hbm, *refs):
        if with_deg:
            agg_out, deg_out, idx_v, mv0, mv1, ones_v, agg_sp, deg_sp, m_sem = refs
        else:
            agg_out, idx_v, mv0, mv1, ones_v, agg_sp, deg_sp, m_sem = refs
        msg_bufs = (mv0, mv1)
        c = lax.axis_index("c")
        s = lax.axis_index("s")
        wid = c * NS + s

        pltpu.sync_copy(aux_hbm.at[pl.ds(0, RPT)], agg_sp.at[pl.ds(s * RPT, RPT)])
        if with_deg:
            pltpu.sync_copy(aux_hbm.at[pl.ds(0, RPT)], deg_sp.at[pl.ds(s * RPT, RPT)])
            pltpu.sync_copy(aux_hbm.at[pl.ds(RPT, GW)], ones_v)
        pltpu.sync_copy(dst_hbm.at[pl.ds(wid * WPW, WPW)], idx_v)
        plsc.subcore_barrier()

        n_chunks = WPW // CH

        def fetch(t, b):
            return pltpu.async_copy(
                msg_hbm.at[pl.ds((wid * WPW + t * CH) * GW, CH * GW)],
                msg_bufs[b],
                m_sem,
            )

        ld = fetch(0, 0)
        for t in range(n_chunks):
            b = t % 2
            ld.wait()
            if t + 1 < n_chunks:
                ld = fetch(t + 1, 1 - b)
            for j in range(CH):
                pltpu.sync_copy(
                    msg_bufs[b].at[pl.ds(j * GW, GW)],
                    agg_sp.at[idx_v.at[t * CH + j]],
                    add=True,
                )
                if with_deg:
                    pltpu.sync_copy(
                        ones_v, deg_sp.at[idx_v.at[t * CH + j]], add=True
                    )

        plsc.subcore_barrier()
        pltpu.sync_copy(
            agg_sp.at[pl.ds(s * RPT, RPT)],
            agg_out.at[pl.ds(c * N_SP + s * RPT, RPT)],
        )
        if with_deg:
            pltpu.sync_copy(
                deg_sp.at[pl.ds(s * RPT, RPT)],
                deg_out.at[pl.ds(c * N_SP + s * RPT, RPT)],
            )

    return kern


def _tc_proj(x8, w8p, b_tile):
    def body(x_ref, w_ref, b_ref, o_ref):
        o_ref[...] = (
            jnp.dot(x_ref[...], w_ref[...], preferred_element_type=jnp.float32)
            + b_ref[...]
        )

    return pl.pallas_call(
        body, out_shape=jax.ShapeDtypeStruct((N * HIDDEN // 128, 128), jnp.float32)
    )(x8, w8p, b_tile)


_BE = 2560
_N_REAL_BLK = E // _BE


_PK = 8
_BR = _BE // _PK
_PW = _PK * HH


def _tc_msg(ef8, hs_p, w8, b8, S8, T8):
    def body(ef_ref, hs_ref, w_ref, b_ref, s_ref, t_ref, o_ref):
        ew_u = jnp.maximum(
            jnp.dot(ef_ref[...], w_ref[...], preferred_element_type=jnp.float32)
            + b_ref[...],
            0.0,
        )
        ew = ew_u.reshape(_BR, _PW)
        rep = jnp.dot(
            hs_ref[...].astype(jnp.bfloat16), s_ref[...],
            preferred_element_type=jnp.float32,
        )
        prod = (rep * ew).astype(jnp.bfloat16)
        o_ref[...] = jnp.dot(prod, t_ref[...], preferred_element_type=jnp.float32)

    return pl.pallas_call(
        body,
        grid=(E_PAD // _BE,),
        in_specs=[
            pl.BlockSpec(
                (_BE, EDGE_FEATS),
                lambda i: (jnp.minimum(i, _N_REAL_BLK - 1), 0),
            ),
            pl.BlockSpec((_BR, 128), lambda i: (i, 0)),
            pl.BlockSpec((EDGE_FEATS, HH), lambda i: (0, 0)),
            pl.BlockSpec((1, HH), lambda i: (0, 0)),
            pl.BlockSpec((128, _PW), lambda i: (0, 0)),
            pl.BlockSpec((_PW, 128), lambda i: (0, 0)),
        ],
        out_specs=pl.BlockSpec((_BR, 128), lambda i: (i, 0)),
        out_shape=jax.ShapeDtypeStruct((E_PAD * HIDDEN // 128, 128), jnp.float32),
    )(ef8, hs_p, w8, b8, S8, T8)


_NPK = N * HIDDEN // 128


def _tc_norm1(a0, a1, d0, d1, b_tile):
    def body(a0_ref, a1_ref, d0_ref, d1_ref, b_ref, h_ref, inv_ref):
        inv = 1.0 / jnp.maximum(d0_ref[...] + d1_ref[...], 1.0)
        inv_ref[...] = inv
        h_ref[...] = jnp.maximum((a0_ref[...] + a1_ref[...]) * inv + b_ref[...], 0.0)

    return pl.pallas_call(
        body,
        out_shape=(
            jax.ShapeDtypeStruct((_NPK, 128), jnp.float32),
            jax.ShapeDtypeStruct((_NPK, 128), jnp.float32),
        ),
    )(a0, a1, d0, d1, b_tile)


def _tc_norm2(a0, a1, inv, b_tile):
    def body(a0_ref, a1_ref, inv_ref, b_ref, h_ref):
        h_ref[...] = jnp.maximum(
            (a0_ref[...] + a1_ref[...]) * inv_ref[...] + b_ref[...], 0.0
        )

    return pl.pallas_call(
        body, out_shape=jax.ShapeDtypeStruct((_NPK, 128), jnp.float32)
    )(a0, a1, inv, b_tile)


_S_NP = np.kron(np.eye(HIDDEN, dtype=np.float32), np.ones((1, HIDDEN), np.float32))
_T_NP = np.tile(np.eye(HIDDEN, dtype=np.float32), (HIDDEN, 1))
_S8_NP = np.kron(np.eye(_PK, dtype=np.float32), _S_NP)
_T8_NP = np.kron(np.eye(_PK, dtype=np.float32), _T_NP)


def kernel(node_features, edge_index, edge_features, node_proj_w, node_proj_b,
           edge_nn_w, edge_nn_b, conv_bias_0, conv_bias_1):
    pad = E_PAD - E
    src = jnp.concatenate([edge_index[0], jnp.zeros((pad,), jnp.int32)])
    src = src.reshape(NW * WPW, GW)
    dst = jnp.concatenate([edge_index[1], jnp.full((pad,), N, jnp.int32)])
    dst = dst.reshape(NW * WPW, GW)
    aux = jnp.concatenate(
        [jnp.zeros((RPT, HIDDEN), jnp.float32), jnp.ones((GW, HIDDEN), jnp.float32)]
    )
    S8 = jnp.asarray(_S8_NP, dtype=jnp.bfloat16)
    T8 = jnp.asarray(_T8_NP, dtype=jnp.bfloat16)
    bnn = edge_nn_b.reshape(1, HH)
    eye8 = jnp.eye(_PK, dtype=jnp.float32)
    w8p = jnp.kron(eye8, node_proj_w)
    bp_tile = jnp.tile(node_proj_b, _PK).reshape(1, 128)
    b0_tile = jnp.tile(conv_bias_0, _PK).reshape(1, 128)
    b1_tile = jnp.tile(conv_bias_1, _PK).reshape(1, 128)
    x8 = node_features.reshape(N // _PK, _PK * IN_FEATS)

    h_p = _tc_proj(x8, w8p, bp_tile)

    def edge_stage(h_packed):
        h_tab = h_packed.reshape(N, HIDDEN)
        hs_p = _sc_gather(h_tab, src).reshape(E_PAD * HIDDEN // 128, 128)
        msg_p = _tc_msg(edge_features, hs_p, edge_nn_w, bnn, S8, T8)
        return msg_p.reshape(E_PAD, HIDDEN)

    npk_sp = N_SP * HIDDEN // 128

    def parts(p):
        pk = p.reshape(NC * npk_sp, 128)
        return pk[:_NPK], pk[npk_sp : npk_sp + _NPK]

    msg = edge_stage(h_p)
    agg_p, deg_p = _make_sc_scatter(True)(msg, dst, aux)
    a0, a1 = parts(agg_p)
    d0, d1 = parts(deg_p)
    h_p, invdeg = _tc_norm1(a0, a1, d0, d1, b0_tile)

    msg = edge_stage(h_p)
    agg_p = _make_sc_scatter(False)(msg, dst, aux)
    a0, a1 = parts(agg_p)
    h_p = _tc_norm2(a0, a1, invdeg, b1_tile)
    return h_p.reshape(N, HIDDEN)


# speedup vs baseline: 6.7870x; 1.0783x over previous
import functools

import jax
import jax.numpy as jnp
import numpy as np
from jax import lax
from jax.experimental import pallas as pl
from jax.experimental.pallas import tpu as pltpu
from jax.experimental.pallas import tpu_sc as plsc

N = 10000
E = 320000
IN_FEATS = 128
EDGE_FEATS = 4
HIDDEN = 16
HH = HIDDEN * HIDDEN

NC = 2
NS = 16
NW = NC * NS
GW = 128
WPW = 80
E_PAD = NW * WPW * GW
RW = E // GW
N_SP = N
RPT = N_SP // NS
CH = 8

@functools.cache
def _mesh():
    return plsc.VectorSubcoreMesh(
        core_axis_name="c", subcore_axis_name="s", num_cores=NC, num_subcores=NS
    )


def _sc_gather(table, idx2d):
    @functools.partial(
        pl.kernel,
        out_type=jax.ShapeDtypeStruct((E_PAD, HIDDEN), jnp.float32),
        mesh=_mesh(),
        scratch_types=[
            pltpu.VMEM((WPW, GW), jnp.int32),
            pltpu.VMEM((CH * GW, HIDDEN), jnp.float32),
            pltpu.VMEM((CH * GW, HIDDEN), jnp.float32),
            pltpu.SemaphoreType.DMA,
            pltpu.SemaphoreType.DMA,
        ],
        compiler_params=pltpu.CompilerParams(use_tc_tiling_on_sc=False),
    )
    def kern(tab_hbm, idx_hbm, out_hbm, idx_v, gbuf0, gbuf1, g_sem, w_sem):
        gbufs = (gbuf0, gbuf1)
        c = lax.axis_index("c")
        s = lax.axis_index("s")
        wid = c * NS + s
        pltpu.sync_copy(idx_hbm.at[pl.ds(wid * WPW, WPW)], idx_v)
        n_chunks = WPW // CH
        wbs = [None] * n_chunks
        for t in range(n_chunks):
            b = t % 2
            if t >= 2:
                wbs[t - 2].wait()
            gs = [
                pltpu.async_copy(
                    tab_hbm.at[idx_v.at[t * CH + j]],
                    gbufs[b].at[pl.ds(j * GW, GW)],
                    g_sem,
                )
                for j in range(CH)
            ]
            for g in gs:
                g.wait()
            wbs[t] = pltpu.async_copy(
                gbufs[b],
                out_hbm.at[pl.ds((wid * WPW + t * CH) * GW, CH * GW)],
                w_sem,
            )
        wbs[n_chunks - 2].wait()
        wbs[n_chunks - 1].wait()

    return kern(table, idx2d)


@functools.cache
def _make_sc_scatter(with_deg):
    n_out = 2 if with_deg else 1
    out_type = tuple(
        jax.ShapeDtypeStruct((NC * N_SP, HIDDEN), jnp.float32) for _ in range(n_out)
    )
    if n_out == 1:
        out_type = out_type[0]

    scratch_types = [
        pltpu.VMEM((WPW, GW), jnp.int32),
        pltpu.VMEM((CH * GW, HIDDEN), jnp.float32),
        pltpu.VMEM((CH * GW, HIDDEN), jnp.float32),
        pltpu.VMEM((GW, HIDDEN), jnp.float32),
        pltpu.VMEM_SHARED((N_SP, HIDDEN), jnp.float32),
        pltpu.VMEM_SHARED((N_SP, HIDDEN), jnp.float32),
        pltpu.SemaphoreType.DMA,
    ]

    @functools.partial(
        pl.kernel, out_type=out_type, mesh=_mesh(), scratch_types=scratch_types,
        compiler_params=pltpu.CompilerParams(use_tc_tiling_on_sc=False),
    )
    def kern(msg_hbm, dst_hbm, aux_hbm, *refs):
        if with_deg:
            agg_out, deg_out, idx_v, mv0, mv1, ones_v, agg_sp, deg_sp, m_sem = refs
        else:
            agg_out, idx_v, mv0, mv1, ones_v, agg_sp, deg_sp, m_sem = refs
        msg_bufs = (mv0, mv1)
        c = lax.axis_index("c")
        s = lax.axis_index("s")
        wid = c * NS + s

        pltpu.sync_copy(aux_hbm.at[pl.ds(0, RPT)], agg_sp.at[pl.ds(s * RPT, RPT)])
        if with_deg:
            pltpu.sync_copy(aux_hbm.at[pl.ds(0, RPT)], deg_sp.at[pl.ds(s * RPT, RPT)])
            pltpu.sync_copy(aux_hbm.at[pl.ds(RPT, GW)], ones_v)
        pltpu.sync_copy(dst_hbm.at[pl.ds(wid * WPW, WPW)], idx_v)
        plsc.subcore_barrier()

        n_chunks = WPW // CH

        def fetch(t, b):
            return pltpu.async_copy(
                msg_hbm.at[pl.ds((wid * WPW + t * CH) * GW, CH * GW)],
                msg_bufs[b],
                m_sem,
            )

        ld = fetch(0, 0)
        for t in range(n_chunks):
            b = t % 2
            ld.wait()
            if t + 1 < n_chunks:
                ld = fetch(t + 1, 1 - b)
            for j in range(CH):
                @pl.when(wid * WPW + t * CH + j < RW)
                def _():
                    pltpu.sync_copy(
                        msg_bufs[b].at[pl.ds(j * GW, GW)],
                        agg_sp.at[idx_v.at[t * CH + j]],
                        add=True,
                    )
                    if with_deg:
                        pltpu.sync_copy(
                            ones_v, deg_sp.at[idx_v.at[t * CH + j]], add=True
                        )

        plsc.subcore_barrier()
        pltpu.sync_copy(
            agg_sp.at[pl.ds(s * RPT, RPT)],
            agg_out.at[pl.ds(c * N_SP + s * RPT, RPT)],
        )
        if with_deg:
            pltpu.sync_copy(
                deg_sp.at[pl.ds(s * RPT, RPT)],
                deg_out.at[pl.ds(c * N_SP + s * RPT, RPT)],
            )

    return kern


def _tc_proj(x8, w8p, b_tile):
    def body(x_ref, w_ref, b_ref, o_ref):
        o_ref[...] = (
            jnp.dot(x_ref[...], w_ref[...], preferred_element_type=jnp.float32)
            + b_ref[...]
        )

    return pl.pallas_call(
        body, out_shape=jax.ShapeDtypeStruct((N * HIDDEN // 128, 128), jnp.float32)
    )(x8, w8p, b_tile)


_BE = 2560
_N_REAL_BLK = E // _BE


_PK = 8
_BR = _BE // _PK
_PW = _PK * HH


def _tc_msg(ef8, hs_p, w8, b8, S8, T8):
    def body(ef_ref, hs_ref, w_ref, b_ref, s_ref, t_ref, o_ref):
        ew_u = jnp.maximum(
            jnp.dot(ef_ref[...], w_ref[...], preferred_element_type=jnp.float32)
            + b_ref[...],
            0.0,
        )
        ew = ew_u.reshape(_BR, _PW)
        rep = jnp.dot(
            hs_ref[...].astype(jnp.bfloat16), s_ref[...],
            preferred_element_type=jnp.float32,
        )
        prod = (rep * ew).astype(jnp.bfloat16)
        o_ref[...] = jnp.dot(prod, t_ref[...], preferred_element_type=jnp.float32)

    return pl.pallas_call(
        body,
        grid=(E_PAD // _BE,),
        in_specs=[
            pl.BlockSpec(
                (_BE, EDGE_FEATS),
                lambda i: (jnp.minimum(i, _N_REAL_BLK - 1), 0),
            ),
            pl.BlockSpec((_BR, 128), lambda i: (i, 0)),
            pl.BlockSpec((EDGE_FEATS, HH), lambda i: (0, 0)),
            pl.BlockSpec((1, HH), lambda i: (0, 0)),
            pl.BlockSpec((128, _PW), lambda i: (0, 0)),
            pl.BlockSpec((_PW, 128), lambda i: (0, 0)),
        ],
        out_specs=pl.BlockSpec((_BR, 128), lambda i: (i, 0)),
        out_shape=jax.ShapeDtypeStruct((E_PAD * HIDDEN // 128, 128), jnp.float32),
    )(ef8, hs_p, w8, b8, S8, T8)


_NPK = N * HIDDEN // 128

_P_SPEC0 = pl.BlockSpec((1, _NPK, 128), lambda i: (0, 0, 0))
_P_SPEC1 = pl.BlockSpec((1, _NPK, 128), lambda i: (1, 0, 0))
_F_SPEC = pl.BlockSpec((_NPK, 128), lambda i: (0, 0))
_B_SPEC = pl.BlockSpec((1, 128), lambda i: (0, 0))


def _tc_norm1(aggpk, degpk, b_tile):
    def body(a0_ref, a1_ref, d0_ref, d1_ref, b_ref, h_ref, inv_ref):
        inv = 1.0 / jnp.maximum(d0_ref[0] + d1_ref[0], 1.0)
        inv_ref[...] = inv
        h_ref[...] = jnp.maximum((a0_ref[0] + a1_ref[0]) * inv + b_ref[...], 0.0)

    return pl.pallas_call(
        body,
        grid=(1,),
        in_specs=[_P_SPEC0, _P_SPEC1, _P_SPEC0, _P_SPEC1, _B_SPEC],
        out_specs=(_F_SPEC, _F_SPEC),
        out_shape=(
            jax.ShapeDtypeStruct((_NPK, 128), jnp.float32),
            jax.ShapeDtypeStruct((_NPK, 128), jnp.float32),
        ),
    )(aggpk, aggpk, degpk, degpk, b_tile)


def _tc_norm2(aggpk, inv, b_tile):
    def body(a0_ref, a1_ref, inv_ref, b_ref, h_ref):
        h_ref[...] = jnp.maximum(
            (a0_ref[0] + a1_ref[0]) * inv_ref[...] + b_ref[...], 0.0
        )

    return pl.pallas_call(
        body,
        grid=(1,),
        in_specs=[_P_SPEC0, _P_SPEC1, _F_SPEC, _B_SPEC],
        out_specs=_F_SPEC,
        out_shape=jax.ShapeDtypeStruct((_NPK, 128), jnp.float32),
    )(aggpk, aggpk, inv, b_tile)


_S_NP = np.kron(np.eye(HIDDEN, dtype=np.float32), np.ones((1, HIDDEN), np.float32))
_T_NP = np.tile(np.eye(HIDDEN, dtype=np.float32), (HIDDEN, 1))
_S8_NP = np.kron(np.eye(_PK, dtype=np.float32), _S_NP)
_T8_NP = np.kron(np.eye(_PK, dtype=np.float32), _T_NP)


def kernel(node_features, edge_index, edge_features, node_proj_w, node_proj_b,
           edge_nn_w, edge_nn_b, conv_bias_0, conv_bias_1):
    pad = E_PAD - E
    src = jnp.concatenate([edge_index[0], jnp.zeros((pad,), jnp.int32)])
    src = src.reshape(NW * WPW, GW)
    dst = jnp.concatenate([edge_index[1], jnp.zeros((pad,), jnp.int32)])
    dst = dst.reshape(NW * WPW, GW)
    aux = jnp.concatenate(
        [jnp.zeros((RPT, HIDDEN), jnp.float32), jnp.ones((GW, HIDDEN), jnp.float32)]
    )
    S8 = jnp.asarray(_S8_NP, dtype=jnp.bfloat16)
    T8 = jnp.asarray(_T8_NP, dtype=jnp.bfloat16)
    bnn = edge_nn_b.reshape(1, HH)
    eye8 = jnp.eye(_PK, dtype=jnp.float32)
    w8p = jnp.kron(eye8, node_proj_w)
    bp_tile = jnp.tile(node_proj_b, _PK).reshape(1, 128)
    b0_tile = jnp.tile(conv_bias_0, _PK).reshape(1, 128)
    b1_tile = jnp.tile(conv_bias_1, _PK).reshape(1, 128)
    x8 = node_features.reshape(N // _PK, _PK * IN_FEATS)

    h_p = _tc_proj(x8, w8p, bp_tile)

    def edge_stage(h_packed):
        h_tab = h_packed.reshape(N, HIDDEN)
        hs_p = _sc_gather(h_tab, src).reshape(E_PAD * HIDDEN // 128, 128)
        msg_p = _tc_msg(edge_features, hs_p, edge_nn_w, bnn, S8, T8)
        return msg_p.reshape(E_PAD, HIDDEN)

    def packed(p):
        return p.reshape(NC, _NPK, 128)

    msg = edge_stage(h_p)
    agg_p, deg_p = _make_sc_scatter(True)(msg, dst, aux)
    h_p, invdeg = _tc_norm1(packed(agg_p), packed(deg_p), b0_tile)

    msg = edge_stage(h_p)
    agg_p = _make_sc_scatter(False)(msg, dst, aux)
    h_p = _tc_norm2(packed(agg_p), invdeg, b1_tile)
    return h_p.reshape(N, HIDDEN)
